# baseline (speedup 1.0000x reference)
import functools

import jax
import jax.numpy as jnp
from jax.experimental import pallas as pl
from jax.experimental.pallas import tpu as pltpu

_SLOPE = 0.01


def _fused_gcn_kernel(adj_ref, x_ref, w_ref, b_ref, o_ref, *, nb, cin, cout):
    M = adj_ref.shape[0]
    adj = adj_ref[...].astype(jnp.bfloat16)
    x2 = x_ref[...].astype(jnp.bfloat16).reshape(M, nb * cin)
    y = jnp.dot(adj, x2, preferred_element_type=jnp.float32)
    y2 = y.astype(jnp.bfloat16).reshape(M * nb, cin)
    w = w_ref[...].astype(jnp.bfloat16)
    h = jnp.dot(y2, w, preferred_element_type=jnp.float32) + b_ref[...]
    o_ref[...] = jnp.maximum(h, _SLOPE * h).reshape(M, nb, cout)


@jax.jit
def kernel(x, adj, weight, bias):
    M, B, Cin = x.shape
    Cout = weight.shape[1]

    x = x.astype(jnp.float32)
    adj = adj.astype(jnp.float32)
    weight = weight.astype(jnp.float32)
    bias2 = bias.astype(jnp.float32).reshape(1, Cout)

    nb = 8

    out = pl.pallas_call(
        functools.partial(_fused_gcn_kernel, nb=nb, cin=Cin, cout=Cout),
        out_shape=jax.ShapeDtypeStruct((M, B, Cout), jnp.float32),
        grid=(B // nb,),
        in_specs=[
            pl.BlockSpec((M, M), lambda j: (0, 0)),
            pl.BlockSpec((M, nb, Cin), lambda j: (0, j, 0)),
            pl.BlockSpec((Cin, Cout), lambda j: (0, 0)),
            pl.BlockSpec((1, Cout), lambda j: (0, 0)),
        ],
        out_specs=pl.BlockSpec((M, nb, Cout), lambda j: (0, j, 0)),
        compiler_params=pltpu.CompilerParams(
            dimension_semantics=("parallel",)),
    )(adj, x, weight, bias2)

    return out


# speedup vs baseline: 1.1876x; 1.1876x over previous
import functools

import jax
import jax.numpy as jnp
from jax.experimental import pallas as pl
from jax.experimental.pallas import tpu as pltpu

_SLOPE = 0.01


def _fused_gcn_kernel(adj_ref, x_ref, w_ref, b_ref, o_ref, *, nb, cin, cout):
    M = adj_ref.shape[0]
    adj = adj_ref[...].astype(jnp.bfloat16)
    x2 = x_ref[...].astype(jnp.bfloat16).reshape(M, nb * cin)
    y = jnp.dot(adj, x2, preferred_element_type=jnp.float32)
    y2 = y.astype(jnp.bfloat16).reshape(M * nb, cin)
    w = w_ref[...].astype(jnp.bfloat16)
    h = jnp.dot(y2, w, preferred_element_type=jnp.float32) + b_ref[...]
    o_ref[...] = jnp.maximum(h, _SLOPE * h).reshape(M, nb, cout)


@jax.jit
def kernel(x, adj, weight, bias):
    M, B, Cin = x.shape
    Cout = weight.shape[1]

    x = x.astype(jnp.float32)
    adj = adj.astype(jnp.float32)
    weight = weight.astype(jnp.float32)
    bias2 = bias.astype(jnp.float32).reshape(1, Cout)

    nb = 16 if B % 16 == 0 else 8

    out = pl.pallas_call(
        functools.partial(_fused_gcn_kernel, nb=nb, cin=Cin, cout=Cout),
        out_shape=jax.ShapeDtypeStruct((M, B, Cout), jnp.float32),
        grid=(B // nb,),
        in_specs=[
            pl.BlockSpec((M, M), lambda j: (0, 0)),
            pl.BlockSpec((M, nb, Cin), lambda j: (0, j, 0)),
            pl.BlockSpec((Cin, Cout), lambda j: (0, 0)),
            pl.BlockSpec((1, Cout), lambda j: (0, 0)),
        ],
        out_specs=pl.BlockSpec((M, nb, Cout), lambda j: (0, j, 0)),
        compiler_params=pltpu.CompilerParams(
            dimension_semantics=("parallel",)),
    )(adj, x, weight, bias2)

    return out
